# speedup vs baseline: 10.9281x; 10.9281x over previous
import functools

import jax
import jax.numpy as jnp
from jax import lax
from jax.experimental import pallas as pl
from jax.experimental.pallas import tpu as pltpu
from jax.experimental.pallas import tpu_sc as plsc

NUM_DAGS = 100000
D = 16
BATCH = 16384
ROW = D * D

NC = 2
NS = 16
NW = NC * NS
B_PER_W = BATCH // NW
CHUNK = 128
NCHUNK = B_PER_W // CHUNK


def _make_gather():
    mesh = plsc.VectorSubcoreMesh(core_axis_name="c", subcore_axis_name="s")

    @functools.partial(
        pl.kernel,
        out_type=jax.ShapeDtypeStruct((BATCH, ROW), jnp.float32),
        mesh=mesh,
        scratch_types=[
            pltpu.VMEM((NCHUNK, CHUNK), jnp.int32),
            pltpu.VMEM((CHUNK, ROW), jnp.float32),
            pltpu.VMEM((CHUNK, ROW), jnp.float32),
            pltpu.SemaphoreType.DMA,
            pltpu.SemaphoreType.DMA,
        ],
    )
    def gather(table_hbm, idx_hbm, out_hbm, idx_v, rows0, rows1, gsem, wsem):
        wid = lax.axis_index("s") * NC + lax.axis_index("c")
        base = wid * B_PER_W
        pltpu.sync_copy(idx_hbm.at[wid], idx_v)
        bufs = (rows0, rows1)
        pending_writes = []
        g0 = pltpu.async_copy(table_hbm.at[idx_v.at[0]], bufs[0], gsem)
        for c in range(NCHUNK):
            buf = bufs[c % 2]
            if c == 0:
                g0.wait()
            else:
                pltpu.async_copy(table_hbm.at[idx_v.at[c]], buf, gsem).wait()
            pending_writes.append(pltpu.async_copy(
                buf, out_hbm.at[pl.ds(base + c * CHUNK, CHUNK)], wsem))
            if c >= 1:
                pending_writes[c - 1].wait()
        pending_writes[-1].wait()

    return gather


_gather_kernel = _make_gather()


def kernel(w, idx):
    table = w.reshape(NUM_DAGS, ROW)
    idx3 = idx.reshape(NW, NCHUNK, CHUNK)
    out = _gather_kernel(table, idx3)
    return out.reshape(BATCH, D, D)


# speedup vs baseline: 11.1569x; 1.0209x over previous
import functools

import jax
import jax.numpy as jnp
from jax import lax
from jax.experimental import pallas as pl
from jax.experimental.pallas import tpu as pltpu
from jax.experimental.pallas import tpu_sc as plsc

NUM_DAGS = 100000
D = 16
BATCH = 16384

TP = D * D
TN = NUM_DAGS

NC = 2
NS = 16
NW = NC * NS
P_PER_W = TP // NW
OUTCH = 4096
NOUTCH = BATCH // OUTCH
L = 16


def _make_gather():
    mesh = plsc.VectorSubcoreMesh(core_axis_name="c", subcore_axis_name="s")

    @functools.partial(
        pl.kernel,
        out_type=jax.ShapeDtypeStruct((TP, BATCH), jnp.float32),
        mesh=mesh,
        compiler_params=pltpu.CompilerParams(needs_layout_passes=False),
        scratch_types=[
            pltpu.VMEM((BATCH,), jnp.int32),
            pltpu.VMEM((TN,), jnp.float32),
            pltpu.VMEM((OUTCH,), jnp.float32),
            pltpu.VMEM((OUTCH,), jnp.float32),
            pltpu.SemaphoreType.DMA((2,)),
        ],
    )
    def gather(table_hbm, idx_hbm, out_hbm, idx_v, row_v, out_a, out_b, wsem):
        wid = lax.axis_index("s") * NC + lax.axis_index("c")
        pltpu.sync_copy(idx_hbm, idx_v)
        obufs = (out_a, out_b)
        prev = [None, None]
        for i in range(P_PER_W):
            p = wid * P_PER_W + i
            pltpu.sync_copy(table_hbm.at[p], row_v)
            for k in range(NOUTCH):
                s = k % 2
                obuf = obufs[s]
                if prev[s] is not None:
                    prev[s].wait()

                def body(g, carry, base=k * OUTCH, obuf=obuf):
                    iv = idx_v[pl.ds(base + g * L, L)]
                    obuf[pl.ds(g * L, L)] = plsc.load_gather(row_v, [iv])
                    return carry

                lax.fori_loop(0, OUTCH // L, body, 0)
                prev[s] = pltpu.async_copy(
                    obuf, out_hbm.at[p, pl.ds(k * OUTCH, OUTCH)], wsem.at[s])
        for s in range(2):
            if prev[s] is not None:
                prev[s].wait()

    return gather


_gather_kernel = _make_gather()


def kernel(w, idx):
    table_t = w.transpose(1, 2, 0).reshape(TP, TN)
    out_t = _gather_kernel(table_t, idx)
    return out_t.reshape(D, D, BATCH).transpose(2, 0, 1)


# speedup vs baseline: 19.2465x; 1.7251x over previous
import functools

import jax
import jax.numpy as jnp
from jax import lax
from jax.experimental import pallas as pl
from jax.experimental.pallas import tpu as pltpu
from jax.experimental.pallas import tpu_sc as plsc

NUM_DAGS = 100000
D = 16
BATCH = 16384

TP = D * D
TN = NUM_DAGS

NC = 2
NS = 16
NW = NC * NS
P_PER_W = TP // NW
OUTCH = 4096
NOUTCH = BATCH // OUTCH
L = 16


def _make_gather():
    mesh = plsc.VectorSubcoreMesh(core_axis_name="c", subcore_axis_name="s")

    @functools.partial(
        pl.kernel,
        out_type=jax.ShapeDtypeStruct((TP, BATCH), jnp.float32),
        mesh=mesh,
        compiler_params=pltpu.CompilerParams(needs_layout_passes=False),
        scratch_types=[
            pltpu.VMEM((BATCH,), jnp.int32),
            pltpu.VMEM((TN,), jnp.float32),
            pltpu.VMEM((OUTCH,), jnp.float32),
            pltpu.VMEM((OUTCH,), jnp.float32),
            pltpu.SemaphoreType.DMA((2,)),
        ],
    )
    def gather(table_hbm, idx_hbm, out_hbm, idx_v, row_v, out_a, out_b, wsem):
        wid = lax.axis_index("s") * NC + lax.axis_index("c")
        pltpu.sync_copy(idx_hbm, idx_v)
        obufs = (out_a, out_b)
        prev = [None, None]
        for i in range(P_PER_W):
            p = wid * P_PER_W + i
            pltpu.sync_copy(table_hbm.at[p], row_v)
            for k in range(NOUTCH):
                s = k % 2
                obuf = obufs[s]
                if prev[s] is not None:
                    prev[s].wait()

                base = k * OUTCH

                @plsc.parallel_loop(0, OUTCH // L, unroll=8)
                def _(g, base=base, obuf=obuf):
                    iv = idx_v[pl.ds(base + g * L, L)]
                    obuf[pl.ds(g * L, L)] = plsc.load_gather(row_v, [iv])
                prev[s] = pltpu.async_copy(
                    obuf, out_hbm.at[p, pl.ds(k * OUTCH, OUTCH)], wsem.at[s])
        for s in range(2):
            if prev[s] is not None:
                prev[s].wait()

    return gather


_gather_kernel = _make_gather()


def kernel(w, idx):
    table_t = w.transpose(1, 2, 0).reshape(TP, TN)
    out_t = _gather_kernel(table_t, idx)
    return out_t.reshape(D, D, BATCH).transpose(2, 0, 1)
